# baseline (speedup 1.0000x reference)
import functools

import jax
import jax.numpy as jnp
from jax import lax
from jax.experimental import pallas as pl
from jax.experimental.pallas import tpu as pltpu
from jax.experimental.pallas import tpu_sc as plsc

TSL = 512
NTP = 128


def _gelu_exact(x):
    return 0.5 * x * (1.0 + jax.lax.erf(x * 0.7071067811865476))


def _router_kernel(x_ref, wr_ref, br_ref,
                   gates_ref, aux_ref, pos1_ref, pos2_ref, c1_ref, c2_ref,
                   tmap_ref, nact_ref):
    E = wr_ref.shape[1]
    N = x_ref.shape[0]
    x = x_ref[...]
    wr = wr_ref[...]
    lt = jax.lax.dot_general(wr, x, (((0,), (1,)), ((), ())),
                             preferred_element_type=jnp.float32)
    lt = lt + br_ref[...].reshape(E, 1)
    m = jnp.max(lt, axis=0, keepdims=True)
    p = jnp.exp(lt - m)
    p = p / jnp.sum(p, axis=0, keepdims=True)
    gates_ref[...] = p.T
    imp = jnp.sum(p, axis=1, keepdims=True)
    load = jnp.sum((p > 0.0).astype(jnp.float32), axis=1, keepdims=True)
    v = imp * load
    mu = jnp.mean(v)
    var = jnp.sum((v - mu) ** 2) / (E - 1)
    aux_ref[...] = jnp.reshape(var * 0.01, (1, 1))
    neg = jnp.float32(-jnp.inf)
    v1 = jnp.full((1, N), neg, jnp.float32)
    e1 = jnp.zeros((1, N), jnp.float32)
    for e in range(E):
        row = p[e:e + 1, :]
        upd = row > v1
        v1 = jnp.where(upd, row, v1)
        e1 = jnp.where(upd, jnp.float32(e), e1)
    v2 = jnp.full((1, N), neg, jnp.float32)
    e2 = jnp.zeros((1, N), jnp.float32)
    for e in range(E):
        row = p[e:e + 1, :]
        upd = (row > v2) & (e1 != jnp.float32(e))
        v2 = jnp.where(upd, row, v2)
        e2 = jnp.where(upd, jnp.float32(e), e2)
    s = v1 + v2
    w1 = v1 / (s + 1e-9)
    w2 = v2 / (s + 1e-9)
    scale = 1.0 / (w1 + w2 + 1e-9)
    c1_ref[...] = w1 * scale
    c2_ref[...] = w2 * scale
    ind1 = []
    ind2 = []
    for e in range(E):
        fe = jnp.float32(e)
        ind1.append((e1 == fe).astype(jnp.float32))
        ind2.append((e2 == fe).astype(jnp.float32))
    i1 = jnp.concatenate(ind1, axis=0)
    i2 = jnp.concatenate(ind2, axis=0)
    ind = i1 + i2
    ps = ind
    k = 1
    while k < N:
        ps = ps + jnp.concatenate(
            [jnp.zeros((E, k), jnp.float32), ps[:, :N - k]], axis=1)
        k *= 2
    excl = ps - ind
    counts = ps[:, N - 1:N]
    ntiles = jnp.ceil(counts / TSL)
    padded = ntiles * TSL
    inc = padded
    k = 1
    while k < E:
        inc = inc + jnp.concatenate(
            [jnp.zeros((k, 1), jnp.float32), inc[:E - k, :]], axis=0)
        k *= 2
    starts = inc - padded
    pos = starts + excl
    pos1 = jnp.sum(i1 * pos, axis=0, keepdims=True)
    pos2 = jnp.sum(i2 * pos, axis=0, keepdims=True)
    pos1_ref[...] = pos1.astype(jnp.int32)
    pos2_ref[...] = pos2.astype(jnp.int32)
    tstart = (inc - padded) / TSL
    jt = jax.lax.broadcasted_iota(jnp.int32, (E, NTP), 1).astype(jnp.float32)
    sel = (jt >= tstart) & (jt < tstart + ntiles)
    ei = jax.lax.broadcasted_iota(jnp.int32, (E, NTP), 0).astype(jnp.float32)
    tmap = jnp.sum(jnp.where(sel, ei, 0.0), axis=0, keepdims=True)
    tmap_ref[...] = tmap.astype(jnp.int32)
    nact_ref[...] = jnp.sum(ntiles, keepdims=True).reshape(1, 1).astype(jnp.int32)


def _router(flat, Wr, br):
    N, H = flat.shape
    E = Wr.shape[1]
    return pl.pallas_call(
        _router_kernel,
        out_shape=(
            jax.ShapeDtypeStruct((N, E), jnp.float32),
            jax.ShapeDtypeStruct((1, 1), jnp.float32),
            jax.ShapeDtypeStruct((1, N), jnp.int32),
            jax.ShapeDtypeStruct((1, N), jnp.int32),
            jax.ShapeDtypeStruct((1, N), jnp.float32),
            jax.ShapeDtypeStruct((1, N), jnp.float32),
            jax.ShapeDtypeStruct((1, NTP), jnp.int32),
            jax.ShapeDtypeStruct((1, 1), jnp.int32),
        ),
    )(flat, Wr, br.reshape(1, E))


def _sc_dispatch(flat, pos1, pos2, nslot):
    N, H = flat.shape
    info = plsc.get_sparse_core_info()
    nw = info.num_cores * info.num_subcores
    chunk = N // nw
    mesh = plsc.VectorSubcoreMesh(core_axis_name="c", subcore_axis_name="s")

    sub = 32
    nsub = chunk // sub

    @functools.partial(
        pl.kernel, mesh=mesh,
        out_type=jax.ShapeDtypeStruct((nslot, H), jnp.float32),
        scratch_types=[
            pltpu.VMEM((chunk,), jnp.int32),
            pltpu.VMEM((chunk,), jnp.int32),
            pltpu.VMEM((sub, H), jnp.float32),
            pltpu.VMEM((sub, H), jnp.float32),
            pltpu.SemaphoreType.DMA((2,)),
            pltpu.SemaphoreType.DMA((2,)),
        ],
    )
    def k(flat_hbm, p1_hbm, p2_hbm, xg_hbm, i1_v, i2_v, rows0, rows1,
          sem1, sem2):
        wid = lax.axis_index("s") * info.num_cores + lax.axis_index("c")
        base = wid * chunk
        pltpu.sync_copy(p1_hbm.at[pl.ds(base, chunk)], i1_v)
        pltpu.sync_copy(p2_hbm.at[pl.ds(base, chunk)], i2_v)
        bufs = (rows0, rows1)
        pend = [None, None]
        for sb in range(nsub):
            b = sb % 2
            if pend[b] is not None:
                pend[b][0].wait()
                pend[b][1].wait()
            pltpu.sync_copy(flat_hbm.at[pl.ds(base + sb * sub, sub)], bufs[b])
            d1 = pltpu.async_copy(bufs[b],
                                  xg_hbm.at[i1_v.at[pl.ds(sb * sub, sub)]],
                                  sem1.at[b])
            d2 = pltpu.async_copy(bufs[b],
                                  xg_hbm.at[i2_v.at[pl.ds(sb * sub, sub)]],
                                  sem2.at[b])
            pend[b] = (d1, d2)
        for b in range(2):
            if pend[b] is not None:
                pend[b][0].wait()
                pend[b][1].wait()

    return k(flat, pos1, pos2)


def _sc_gather2(y, pos1, pos2):
    nslot, H = y.shape
    N = pos1.shape[0]
    info = plsc.get_sparse_core_info()
    nw = info.num_cores * info.num_subcores
    chunk = N // nw
    mesh = plsc.VectorSubcoreMesh(core_axis_name="c", subcore_axis_name="s")

    sub = 16
    nsub = chunk // sub

    @functools.partial(
        pl.kernel, mesh=mesh,
        out_type=(
            jax.ShapeDtypeStruct((N, H), jnp.float32),
            jax.ShapeDtypeStruct((N, H), jnp.float32),
        ),
        scratch_types=[
            pltpu.VMEM((chunk,), jnp.int32),
            pltpu.VMEM((chunk,), jnp.int32),
            pltpu.VMEM((sub, H), jnp.float32),
            pltpu.VMEM((sub, H), jnp.float32),
            pltpu.VMEM((sub, H), jnp.float32),
            pltpu.VMEM((sub, H), jnp.float32),
            pltpu.SemaphoreType.DMA((2,)),
            pltpu.SemaphoreType.DMA((2,)),
            pltpu.SemaphoreType.DMA((2,)),
            pltpu.SemaphoreType.DMA((2,)),
        ],
    )
    def k(y_hbm, p1_hbm, p2_hbm, a_hbm, b_hbm, i1_v, i2_v, ra0, rb0, ra1, rb1,
          sga, sgb, ssa, ssb):
        wid = lax.axis_index("s") * info.num_cores + lax.axis_index("c")
        base = wid * chunk
        pltpu.sync_copy(p1_hbm.at[pl.ds(base, chunk)], i1_v)
        pltpu.sync_copy(p2_hbm.at[pl.ds(base, chunk)], i2_v)
        abufs = (ra0, ra1)
        bbufs = (rb0, rb1)
        pend_g = [None, None]
        pend_s = [None, None]
        for sb in range(nsub):
            b = sb % 2
            if pend_s[b] is not None:
                pend_s[b][0].wait()
                pend_s[b][1].wait()
            d1 = pltpu.async_copy(y_hbm.at[i1_v.at[pl.ds(sb * sub, sub)]],
                                  abufs[b], sga.at[b])
            d2 = pltpu.async_copy(y_hbm.at[i2_v.at[pl.ds(sb * sub, sub)]],
                                  bbufs[b], sgb.at[b])
            pend_g[b] = (d1, d2)
            pend_g[b][0].wait()
            pend_g[b][1].wait()
            s1 = pltpu.async_copy(abufs[b], a_hbm.at[pl.ds(base + sb * sub, sub)],
                                  ssa.at[b])
            s2 = pltpu.async_copy(bbufs[b], b_hbm.at[pl.ds(base + sb * sub, sub)],
                                  ssb.at[b])
            pend_s[b] = (s1, s2)
        for b in range(2):
            if pend_s[b] is not None:
                pend_s[b][0].wait()
                pend_s[b][1].wait()

    return k(y, pos1, pos2)


NQ = 4


def _mlp_kernel(m_ref, na_ref, xg_ref, *refs):
    w1q = refs[:NQ]
    w2q = refs[NQ:2 * NQ]
    b1_ref, b2_ref, y_ref = refs[2 * NQ], refs[2 * NQ + 1], refs[2 * NQ + 2]
    t = pl.program_id(0)

    @pl.when(t < na_ref[0])
    def _compute():
        x = xg_ref[...].astype(jnp.bfloat16)
        qm = b1_ref.shape[2] // NQ
        y = None
        for q in range(NQ):
            h = (jnp.dot(x, w1q[q][0], preferred_element_type=jnp.float32)
                 + b1_ref[0, :, q * qm:(q + 1) * qm])
            h = _gelu_exact(h).astype(jnp.bfloat16)
            part = jnp.dot(h, w2q[q][0], preferred_element_type=jnp.float32)
            y = part if y is None else y + part
        y_ref[...] = y + b2_ref[0]


def _grouped_mlp(tmap, nact, xg, W1, b1, W2, b2, nt):
    nslot, H = xg.shape
    E, _, MLPD = W1.shape
    qm = MLPD // NQ
    w1q = [W1[:, :, q * qm:(q + 1) * qm] for q in range(NQ)]
    w2q = [W2[:, q * qm:(q + 1) * qm, :] for q in range(NQ)]
    grid_spec = pltpu.PrefetchScalarGridSpec(
        num_scalar_prefetch=2,
        grid=(nt,),
        in_specs=(
            [pl.BlockSpec((TSL, H), lambda t, m, na: (t, 0))]
            + [pl.BlockSpec((1, H, qm), lambda t, m, na: (m[t], 0, 0))
               for _ in range(NQ)]
            + [pl.BlockSpec((1, qm, H), lambda t, m, na: (m[t], 0, 0))
               for _ in range(NQ)]
            + [pl.BlockSpec((1, 1, MLPD), lambda t, m, na: (m[t], 0, 0)),
               pl.BlockSpec((1, 1, H), lambda t, m, na: (m[t], 0, 0))]
        ),
        out_specs=pl.BlockSpec((TSL, H), lambda t, m, na: (t, 0)),
    )
    return pl.pallas_call(
        _mlp_kernel,
        grid_spec=grid_spec,
        out_shape=jax.ShapeDtypeStruct((nslot, H), jnp.float32),
    )(tmap, nact, xg, *w1q, *w2q, b1, b2)


def _combine_kernel(a_ref, b_ref, c1_ref, c2_ref, out_ref):
    out_ref[...] = c1_ref[...] * a_ref[...] + c2_ref[...] * b_ref[...]


def _combine(A, B, c1, c2):
    N, H = A.shape
    TB = 512
    return pl.pallas_call(
        _combine_kernel,
        grid=(N // TB,),
        in_specs=[
            pl.BlockSpec((TB, H), lambda t: (t, 0)),
            pl.BlockSpec((TB, H), lambda t: (t, 0)),
            pl.BlockSpec((TB, 1), lambda t: (t, 0)),
            pl.BlockSpec((TB, 1), lambda t: (t, 0)),
        ],
        out_specs=pl.BlockSpec((TB, H), lambda t: (t, 0)),
        out_shape=jax.ShapeDtypeStruct((N, H), jnp.float32),
    )(A, B, c1, c2)


def kernel(inputs, Wr, br, W1, b1, W2, b2):
    ns, L, H = inputs.shape
    E = Wr.shape[1]
    MLPD = W1.shape[2]
    GSZ = 512
    N = ns * L
    K = 2
    nt = (K * N) // TSL + E - 1
    nt = ((nt + 1) // 2) * 2
    nslot = nt * TSL
    flat = inputs.reshape(N, H)

    gates, aux, pos1, pos2, c1, c2, tmap, nact = _router(flat, Wr, br)
    p1 = pos1.reshape(N)
    p2 = pos2.reshape(N)
    xg = _sc_dispatch(flat, p1, p2, nslot)
    y = _grouped_mlp(tmap.reshape(NTP), nact.reshape(1), xg,
                     W1.astype(jnp.bfloat16), b1.reshape(E, 1, MLPD),
                     W2.astype(jnp.bfloat16), b2.reshape(E, 1, H), nt)
    A, B = _sc_gather2(y, p1, p2)
    out = _combine(A, B, c1.reshape(N, 1), c2.reshape(N, 1))
    return (out.reshape(ns, L, H), aux[0, 0],
            gates.reshape(N // GSZ, GSZ, E))


# speedup vs baseline: 1.2202x; 1.2202x over previous
import functools

import jax
import jax.numpy as jnp
from jax import lax
from jax.experimental import pallas as pl
from jax.experimental.pallas import tpu as pltpu
from jax.experimental.pallas import tpu_sc as plsc

TSL = 512
NTP = 128


def _gelu_exact(x):
    return 0.5 * x * (1.0 + jax.lax.erf(x * 0.7071067811865476))


def _router_kernel(x_ref, wr_ref, br_ref,
                   gates_ref, aux_ref, pos1_ref, pos2_ref, c1_ref, c2_ref,
                   tmap_ref, nact_ref):
    E = wr_ref.shape[1]
    N = x_ref.shape[0]
    x = x_ref[...]
    wr = wr_ref[...]
    lt = jax.lax.dot_general(wr, x, (((0,), (1,)), ((), ())),
                             preferred_element_type=jnp.float32)
    lt = lt + br_ref[...].reshape(E, 1)
    m = jnp.max(lt, axis=0, keepdims=True)
    p = jnp.exp(lt - m)
    p = p / jnp.sum(p, axis=0, keepdims=True)
    gates_ref[...] = p.T
    imp = jnp.sum(p, axis=1, keepdims=True)
    load = jnp.sum((p > 0.0).astype(jnp.float32), axis=1, keepdims=True)
    v = imp * load
    mu = jnp.mean(v)
    var = jnp.sum((v - mu) ** 2) / (E - 1)
    aux_ref[...] = jnp.reshape(var * 0.01, (1, 1))
    neg = jnp.float32(-jnp.inf)
    v1 = jnp.full((1, N), neg, jnp.float32)
    e1 = jnp.zeros((1, N), jnp.float32)
    for e in range(E):
        row = p[e:e + 1, :]
        upd = row > v1
        v1 = jnp.where(upd, row, v1)
        e1 = jnp.where(upd, jnp.float32(e), e1)
    v2 = jnp.full((1, N), neg, jnp.float32)
    e2 = jnp.zeros((1, N), jnp.float32)
    for e in range(E):
        row = p[e:e + 1, :]
        upd = (row > v2) & (e1 != jnp.float32(e))
        v2 = jnp.where(upd, row, v2)
        e2 = jnp.where(upd, jnp.float32(e), e2)
    s = v1 + v2
    w1 = v1 / (s + 1e-9)
    w2 = v2 / (s + 1e-9)
    scale = 1.0 / (w1 + w2 + 1e-9)
    c1_ref[...] = w1 * scale
    c2_ref[...] = w2 * scale
    ind1 = []
    ind2 = []
    for e in range(E):
        fe = jnp.float32(e)
        ind1.append((e1 == fe).astype(jnp.float32))
        ind2.append((e2 == fe).astype(jnp.float32))
    i1 = jnp.concatenate(ind1, axis=0)
    i2 = jnp.concatenate(ind2, axis=0)
    ind = i1 + i2
    ps = ind
    k = 1
    while k < N:
        ps = ps + jnp.concatenate(
            [jnp.zeros((E, k), jnp.float32), ps[:, :N - k]], axis=1)
        k *= 2
    excl = ps - ind
    counts = ps[:, N - 1:N]
    ntiles = jnp.ceil(counts / TSL)
    padded = ntiles * TSL
    inc = padded
    k = 1
    while k < E:
        inc = inc + jnp.concatenate(
            [jnp.zeros((k, 1), jnp.float32), inc[:E - k, :]], axis=0)
        k *= 2
    starts = inc - padded
    pos = starts + excl
    pos1 = jnp.sum(i1 * pos, axis=0, keepdims=True)
    pos2 = jnp.sum(i2 * pos, axis=0, keepdims=True)
    pos1_ref[...] = pos1.astype(jnp.int32)
    pos2_ref[...] = pos2.astype(jnp.int32)
    tstart = (inc - padded) / TSL
    jt = jax.lax.broadcasted_iota(jnp.int32, (E, NTP), 1).astype(jnp.float32)
    sel = (jt >= tstart) & (jt < tstart + ntiles)
    ei = jax.lax.broadcasted_iota(jnp.int32, (E, NTP), 0).astype(jnp.float32)
    tmap = jnp.sum(jnp.where(sel, ei, 0.0), axis=0, keepdims=True)
    tmap_ref[...] = tmap.astype(jnp.int32)
    nact_ref[...] = jnp.sum(ntiles, keepdims=True).reshape(1, 1).astype(jnp.int32)


def _router(flat, Wr, br):
    N, H = flat.shape
    E = Wr.shape[1]
    return pl.pallas_call(
        _router_kernel,
        out_shape=(
            jax.ShapeDtypeStruct((N, E), jnp.float32),
            jax.ShapeDtypeStruct((1, 1), jnp.float32),
            jax.ShapeDtypeStruct((1, N), jnp.int32),
            jax.ShapeDtypeStruct((1, N), jnp.int32),
            jax.ShapeDtypeStruct((1, N), jnp.float32),
            jax.ShapeDtypeStruct((1, N), jnp.float32),
            jax.ShapeDtypeStruct((1, NTP), jnp.int32),
            jax.ShapeDtypeStruct((1, 1), jnp.int32),
        ),
    )(flat, Wr, br.reshape(1, E))


def _sc_dispatch(flat, pos1, pos2, nslot):
    N, H = flat.shape
    info = plsc.get_sparse_core_info()
    nw = info.num_cores * info.num_subcores
    chunk = N // nw
    mesh = plsc.VectorSubcoreMesh(core_axis_name="c", subcore_axis_name="s")

    sub = 32
    nsub = chunk // sub

    @functools.partial(
        pl.kernel, mesh=mesh,
        out_type=jax.ShapeDtypeStruct((nslot, H), jnp.float32),
        scratch_types=[
            pltpu.VMEM((chunk,), jnp.int32),
            pltpu.VMEM((chunk,), jnp.int32),
            pltpu.VMEM((sub, H), jnp.float32),
            pltpu.VMEM((sub, H), jnp.float32),
            pltpu.SemaphoreType.DMA((2,)),
            pltpu.SemaphoreType.DMA((2,)),
        ],
    )
    def k(flat_hbm, p1_hbm, p2_hbm, xg_hbm, i1_v, i2_v, rows0, rows1,
          sem1, sem2):
        wid = lax.axis_index("s") * info.num_cores + lax.axis_index("c")
        base = wid * chunk
        pltpu.sync_copy(p1_hbm.at[pl.ds(base, chunk)], i1_v)
        pltpu.sync_copy(p2_hbm.at[pl.ds(base, chunk)], i2_v)
        bufs = (rows0, rows1)
        pend = [None, None]
        for sb in range(nsub):
            b = sb % 2
            if pend[b] is not None:
                pend[b][0].wait()
                pend[b][1].wait()
            pltpu.sync_copy(flat_hbm.at[pl.ds(base + sb * sub, sub)], bufs[b])
            d1 = pltpu.async_copy(bufs[b],
                                  xg_hbm.at[i1_v.at[pl.ds(sb * sub, sub)]],
                                  sem1.at[b])
            d2 = pltpu.async_copy(bufs[b],
                                  xg_hbm.at[i2_v.at[pl.ds(sb * sub, sub)]],
                                  sem2.at[b])
            pend[b] = (d1, d2)
        for b in range(2):
            if pend[b] is not None:
                pend[b][0].wait()
                pend[b][1].wait()

    return k(flat, pos1, pos2)


def _sc_gather2(y, pos1, pos2):
    nslot, H = y.shape
    N = pos1.shape[0]
    info = plsc.get_sparse_core_info()
    nw = info.num_cores * info.num_subcores
    chunk = N // nw
    mesh = plsc.VectorSubcoreMesh(core_axis_name="c", subcore_axis_name="s")

    sub = 16
    nsub = chunk // sub

    @functools.partial(
        pl.kernel, mesh=mesh,
        out_type=(
            jax.ShapeDtypeStruct((N, H), jnp.float32),
            jax.ShapeDtypeStruct((N, H), jnp.float32),
        ),
        scratch_types=[
            pltpu.VMEM((chunk,), jnp.int32),
            pltpu.VMEM((chunk,), jnp.int32),
            pltpu.VMEM((sub, H), jnp.float32),
            pltpu.VMEM((sub, H), jnp.float32),
            pltpu.VMEM((sub, H), jnp.float32),
            pltpu.VMEM((sub, H), jnp.float32),
            pltpu.SemaphoreType.DMA((2,)),
            pltpu.SemaphoreType.DMA((2,)),
            pltpu.SemaphoreType.DMA((2,)),
            pltpu.SemaphoreType.DMA((2,)),
        ],
    )
    def k(y_hbm, p1_hbm, p2_hbm, a_hbm, b_hbm, i1_v, i2_v, ra0, rb0, ra1, rb1,
          sga, sgb, ssa, ssb):
        wid = lax.axis_index("s") * info.num_cores + lax.axis_index("c")
        base = wid * chunk
        pltpu.sync_copy(p1_hbm.at[pl.ds(base, chunk)], i1_v)
        pltpu.sync_copy(p2_hbm.at[pl.ds(base, chunk)], i2_v)
        abufs = (ra0, ra1)
        bbufs = (rb0, rb1)
        pend_g = [None, None]
        pend_s = [None, None]
        for sb in range(nsub):
            b = sb % 2
            if pend_s[b] is not None:
                pend_s[b][0].wait()
                pend_s[b][1].wait()
            d1 = pltpu.async_copy(y_hbm.at[i1_v.at[pl.ds(sb * sub, sub)]],
                                  abufs[b], sga.at[b])
            d2 = pltpu.async_copy(y_hbm.at[i2_v.at[pl.ds(sb * sub, sub)]],
                                  bbufs[b], sgb.at[b])
            pend_g[b] = (d1, d2)
            pend_g[b][0].wait()
            pend_g[b][1].wait()
            s1 = pltpu.async_copy(abufs[b], a_hbm.at[pl.ds(base + sb * sub, sub)],
                                  ssa.at[b])
            s2 = pltpu.async_copy(bbufs[b], b_hbm.at[pl.ds(base + sb * sub, sub)],
                                  ssb.at[b])
            pend_s[b] = (s1, s2)
        for b in range(2):
            if pend_s[b] is not None:
                pend_s[b][0].wait()
                pend_s[b][1].wait()

    return k(y, pos1, pos2)


def _mlp_kernel(m_ref, na_ref, xg_ref, w1_ref, b1_ref, w2_ref, b2_ref,
                y_ref):
    t = pl.program_id(0)

    @pl.when(t < na_ref[0])
    def _compute():
        x = xg_ref[...].astype(jnp.bfloat16)
        h = jnp.dot(x, w1_ref[0], preferred_element_type=jnp.float32) + b1_ref[0]
        h = _gelu_exact(h).astype(jnp.bfloat16)
        y = jnp.dot(h, w2_ref[0], preferred_element_type=jnp.float32) + b2_ref[0]
        y_ref[...] = y


def _grouped_mlp(tmap, nact, xg, W1, b1, W2, b2, nt):
    nslot, H = xg.shape
    E, _, MLPD = W1.shape
    grid_spec = pltpu.PrefetchScalarGridSpec(
        num_scalar_prefetch=2,
        grid=(nt,),
        in_specs=[
            pl.BlockSpec((TSL, H), lambda t, m, na: (t, 0)),
            pl.BlockSpec((1, H, MLPD), lambda t, m, na: (m[t], 0, 0)),
            pl.BlockSpec((1, 1, MLPD), lambda t, m, na: (m[t], 0, 0)),
            pl.BlockSpec((1, MLPD, H), lambda t, m, na: (m[t], 0, 0)),
            pl.BlockSpec((1, 1, H), lambda t, m, na: (m[t], 0, 0)),
        ],
        out_specs=pl.BlockSpec((TSL, H), lambda t, m, na: (t, 0)),
    )
    return pl.pallas_call(
        _mlp_kernel,
        grid_spec=grid_spec,
        out_shape=jax.ShapeDtypeStruct((nslot, H), jnp.float32),
    )(tmap, nact, xg, W1, b1, W2, b2)


def _combine_kernel(a_ref, b_ref, c1_ref, c2_ref, out_ref):
    out_ref[...] = c1_ref[...] * a_ref[...] + c2_ref[...] * b_ref[...]


def _combine(A, B, c1, c2):
    N, H = A.shape
    TB = 512
    return pl.pallas_call(
        _combine_kernel,
        grid=(N // TB,),
        in_specs=[
            pl.BlockSpec((TB, H), lambda t: (t, 0)),
            pl.BlockSpec((TB, H), lambda t: (t, 0)),
            pl.BlockSpec((TB, 1), lambda t: (t, 0)),
            pl.BlockSpec((TB, 1), lambda t: (t, 0)),
        ],
        out_specs=pl.BlockSpec((TB, H), lambda t: (t, 0)),
        out_shape=jax.ShapeDtypeStruct((N, H), jnp.float32),
    )(A, B, c1, c2)


def kernel(inputs, Wr, br, W1, b1, W2, b2):
    ns, L, H = inputs.shape
    E = Wr.shape[1]
    MLPD = W1.shape[2]
    GSZ = 512
    N = ns * L
    K = 2
    nt = (K * N) // TSL + E - 1
    nt = ((nt + 1) // 2) * 2
    nslot = nt * TSL
    flat = inputs.reshape(N, H)

    gates, aux, pos1, pos2, c1, c2, tmap, nact = _router(flat, Wr, br)
    p1 = pos1.reshape(N)
    p2 = pos2.reshape(N)
    xg = _sc_dispatch(flat, p1, p2, nslot)
    y = _grouped_mlp(tmap.reshape(NTP), nact.reshape(1), xg,
                     W1.astype(jnp.bfloat16), b1.reshape(E, 1, MLPD),
                     W2.astype(jnp.bfloat16), b2.reshape(E, 1, H), nt)
    A, B = _sc_gather2(y, p1, p2)
    out = _combine(A, B, c1.reshape(N, 1), c2.reshape(N, 1))
    return (out.reshape(ns, L, H), aux[0, 0],
            gates.reshape(N // GSZ, GSZ, E))
